# baseline (device time: 76248 ns/iter reference)
import jax
import jax.numpy as jnp
from jax import lax
from jax.experimental import pallas as pl
from jax.experimental.pallas import tpu as pltpu

N_DEV = 16


def kernel(x, Win0, Wout0, Win1, Wout1, Win2, Wout2):
    b, d_in = x.shape
    _, h_dim = Win0.shape
    _, d_out = Wout0.shape
    chunk = h_dim // N_DEV

    def body(x_ref, win0_ref, wout0_ref, win1_ref, wout1_ref,
             win2_ref, wout2_ref, out_ref,
             acc_ref, rs_recv_ref, hchunk_ref, hfull_ref, xbuf_ref,
             rs_send_sem, rs_recv_sem, ag_send_sem, ag_recv_sems):
        my_i = lax.axis_index("i")

        def layer(xin_ref, win_ref, wout_ref, xout_ref):
            for k in range(N_DEV - 1):
                tgt = (my_i + 1 + k) % N_DEV
                acc_ref[:, pl.ds(tgt * chunk, chunk)] = jnp.dot(
                    xin_ref[:, :], win_ref[:, pl.ds(tgt * chunk, chunk)],
                    preferred_element_type=jnp.float32,
                )
                rdma = pltpu.make_async_remote_copy(
                    src_ref=acc_ref.at[:, pl.ds(tgt * chunk, chunk)],
                    dst_ref=rs_recv_ref.at[my_i],
                    send_sem=rs_send_sem,
                    recv_sem=rs_recv_sem,
                    device_id=(tgt,),
                    device_id_type=pl.DeviceIdType.MESH,
                )
                rdma.start()

            rs_recv_ref[my_i] = jnp.dot(
                xin_ref[:, :], win_ref[:, pl.ds(my_i * chunk, chunk)],
                preferred_element_type=jnp.float32,
            )

            for k in range(N_DEV - 1):
                wait = pltpu.make_async_remote_copy(
                    src_ref=acc_ref.at[:, pl.ds(0, chunk)],
                    dst_ref=rs_recv_ref.at[k],
                    send_sem=rs_send_sem,
                    recv_sem=rs_recv_sem,
                    device_id=(my_i,),
                    device_id_type=pl.DeviceIdType.MESH,
                )
                wait.wait_recv()

            h_me = rs_recv_ref[0]
            for j in range(1, N_DEV):
                h_me = h_me + rs_recv_ref[j]
            hchunk_ref[:, :] = jnp.maximum(h_me, 0.0)

            for k in range(N_DEV - 1):
                wait = pltpu.make_async_remote_copy(
                    src_ref=acc_ref.at[:, pl.ds(0, chunk)],
                    dst_ref=rs_recv_ref.at[k],
                    send_sem=rs_send_sem,
                    recv_sem=rs_recv_sem,
                    device_id=(my_i,),
                    device_id_type=pl.DeviceIdType.MESH,
                )
                wait.wait_send()

            for k in range(N_DEV - 1):
                tgt = (my_i + 1 + k) % N_DEV
                rdma = pltpu.make_async_remote_copy(
                    src_ref=hchunk_ref,
                    dst_ref=hfull_ref.at[:, pl.ds(my_i * chunk, chunk)],
                    send_sem=ag_send_sem,
                    recv_sem=ag_recv_sems.at[my_i],
                    device_id=(tgt,),
                    device_id_type=pl.DeviceIdType.MESH,
                )
                rdma.start()

            xout_ref[:, :] = jnp.dot(
                hchunk_ref[:, :], wout_ref[pl.ds(my_i * chunk, chunk), :],
                preferred_element_type=jnp.float32,
            )

            for k in range(N_DEV - 1):
                src = (my_i + 1 + k) % N_DEV
                wait = pltpu.make_async_remote_copy(
                    src_ref=hchunk_ref,
                    dst_ref=hfull_ref.at[:, pl.ds(src * chunk, chunk)],
                    send_sem=ag_send_sem,
                    recv_sem=ag_recv_sems.at[src],
                    device_id=(my_i,),
                    device_id_type=pl.DeviceIdType.MESH,
                )
                wait.wait_recv()
                xout_ref[:, :] = xout_ref[:, :] + jnp.dot(
                    hfull_ref[:, pl.ds(src * chunk, chunk)],
                    wout_ref[pl.ds(src * chunk, chunk), :],
                    preferred_element_type=jnp.float32,
                )

            for k in range(N_DEV - 1):
                wait = pltpu.make_async_remote_copy(
                    src_ref=hchunk_ref,
                    dst_ref=hfull_ref.at[:, pl.ds(k * chunk, chunk)],
                    send_sem=ag_send_sem,
                    recv_sem=ag_recv_sems.at[k],
                    device_id=(my_i,),
                    device_id_type=pl.DeviceIdType.MESH,
                )
                wait.wait_send()

        layer(x_ref, win0_ref, wout0_ref, xbuf_ref)
        layer(xbuf_ref, win1_ref, wout1_ref, xbuf_ref)
        layer(xbuf_ref, win2_ref, wout2_ref, out_ref)

    return pl.pallas_call(
        body,
        out_shape=jax.ShapeDtypeStruct((b, d_out), jnp.float32),
        in_specs=[pl.BlockSpec(memory_space=pltpu.VMEM)] * 7,
        out_specs=pl.BlockSpec(memory_space=pltpu.VMEM),
        scratch_shapes=[
            pltpu.VMEM((b, h_dim), jnp.float32),
            pltpu.VMEM((N_DEV, b, chunk), jnp.float32),
            pltpu.VMEM((b, chunk), jnp.float32),
            pltpu.VMEM((b, h_dim), jnp.float32),
            pltpu.VMEM((b, d_in), jnp.float32),
            pltpu.SemaphoreType.DMA,
            pltpu.SemaphoreType.DMA,
            pltpu.SemaphoreType.DMA,
            pltpu.SemaphoreType.DMA((N_DEV,)),
        ],
        compiler_params=pltpu.CompilerParams(
            vmem_limit_bytes=100 * 1024 * 1024,
        ),
    )(x, Win0, Wout0, Win1, Wout1, Win2, Wout2)


# device time: 55812 ns/iter; 1.3662x vs baseline; 1.3662x over previous
import jax
import jax.numpy as jnp
from jax import lax
from jax.experimental import pallas as pl
from jax.experimental.pallas import tpu as pltpu

N_DEV = 16
H = 2
N_LAYER = 3


def kernel(x, Win0, Wout0, Win1, Wout1, Win2, Wout2):
    b, d_in = x.shape
    _, h_dim = Win0.shape
    _, d_out = Wout0.shape
    chunk = h_dim // N_DEV
    hb = b // H

    def body(x_ref, win0_ref, wout0_ref, win1_ref, wout1_ref,
             win2_ref, wout2_ref, out_ref,
             acc_ref, rs_recv_ref, hchunk_ref, hfull_ref,
             xbuf_ref,
             rs_send_sems, rs_recv_sems, ag_send_sems, ag_recv_sems):
        my_i = lax.axis_index("i")
        wins = [win0_ref, win1_ref, win2_ref]
        wouts = [wout0_ref, wout1_ref, wout2_ref]
        xins = [x_ref, xbuf_ref, xbuf_ref]
        xouts = [xbuf_ref, xbuf_ref, out_ref]

        barrier_sem = pltpu.get_barrier_semaphore()
        for k in range(N_DEV - 1):
            peer = (my_i + 1 + k) % N_DEV
            pl.semaphore_signal(
                barrier_sem, inc=1,
                device_id=(peer,),
                device_id_type=pl.DeviceIdType.MESH,
            )
        pl.semaphore_wait(barrier_sem, N_DEV - 1)

        def rs_descriptor(h, tgt):
            return pltpu.make_async_remote_copy(
                src_ref=acc_ref.at[pl.ds(h * hb, hb),
                                   pl.ds(tgt * chunk, chunk)],
                dst_ref=rs_recv_ref.at[my_i].at[pl.ds(h * hb, hb)],
                send_sem=rs_send_sems.at[h],
                recv_sem=rs_recv_sems.at[h],
                device_id=(tgt,),
                device_id_type=pl.DeviceIdType.MESH,
            )

        def ag_descriptor(h, tgt):
            return pltpu.make_async_remote_copy(
                src_ref=hchunk_ref.at[pl.ds(h * hb, hb)],
                dst_ref=hfull_ref.at[pl.ds(h * hb, hb),
                                     pl.ds(my_i * chunk, chunk)],
                send_sem=ag_send_sems.at[h],
                recv_sem=ag_recv_sems.at[h],
                device_id=(tgt,),
                device_id_type=pl.DeviceIdType.MESH,
            )

        def s1(l, h):
            acc_ref[pl.ds(h * hb, hb), :] = jnp.dot(
                xins[l][pl.ds(h * hb, hb), :], wins[l][:, :],
                preferred_element_type=jnp.float32,
            ).astype(jnp.bfloat16)
            for k in range(N_DEV - 1):
                tgt = (my_i + 1 + k) % N_DEV
                rs_descriptor(h, tgt).start()
            rs_recv_ref[my_i, pl.ds(h * hb, hb)] = (
                acc_ref[pl.ds(h * hb, hb), pl.ds(my_i * chunk, chunk)]
            )

        def s2(l, h):
            for k in range(N_DEV - 1):
                rs_descriptor(h, my_i).wait_recv()
            h_me = jnp.sum(
                rs_recv_ref[:, pl.ds(h * hb, hb), :].astype(jnp.float32),
                axis=0,
            )
            hchunk_ref[pl.ds(h * hb, hb), :] = (
                jnp.maximum(h_me, 0.0).astype(jnp.bfloat16)
            )
            for k in range(N_DEV - 1):
                tgt = (my_i + 1 + k) % N_DEV
                ag_descriptor(h, tgt).start()
            hfull_ref[pl.ds(h * hb, hb), pl.ds(my_i * chunk, chunk)] = (
                hchunk_ref[pl.ds(h * hb, hb), :]
            )
            for k in range(N_DEV - 1):
                rs_descriptor(h, my_i).wait_send()

        def s3(l, h):
            for k in range(N_DEV - 1):
                ag_descriptor(h, my_i).wait_recv()
            xouts[l][pl.ds(h * hb, hb), :] = jnp.dot(
                hfull_ref[pl.ds(h * hb, hb), :].astype(jnp.float32),
                wouts[l][:, :],
                preferred_element_type=jnp.float32,
            )
            for k in range(N_DEV - 1):
                ag_descriptor(h, my_i).wait_send()

        for h in range(H):
            s1(0, h)
        for l in range(N_LAYER):
            for h in range(H):
                s2(l, h)
            for h in range(H):
                s3(l, h)
                if l + 1 < N_LAYER:
                    s1(l + 1, h)

    return pl.pallas_call(
        body,
        out_shape=jax.ShapeDtypeStruct((b, d_out), jnp.float32),
        in_specs=[pl.BlockSpec(memory_space=pltpu.VMEM)] * 7,
        out_specs=pl.BlockSpec(memory_space=pltpu.VMEM),
        scratch_shapes=[
            pltpu.VMEM((b, h_dim), jnp.bfloat16),
            pltpu.VMEM((N_DEV, b, chunk), jnp.bfloat16),
            pltpu.VMEM((b, chunk), jnp.bfloat16),
            pltpu.VMEM((b, h_dim), jnp.bfloat16),
            pltpu.VMEM((b, d_in), jnp.float32),
            pltpu.SemaphoreType.DMA((H,)),
            pltpu.SemaphoreType.DMA((H,)),
            pltpu.SemaphoreType.DMA((H,)),
            pltpu.SemaphoreType.DMA((H,)),
        ],
        compiler_params=pltpu.CompilerParams(
            vmem_limit_bytes=100 * 1024 * 1024,
            collective_id=0,
        ),
    )(x, Win0, Wout0, Win1, Wout1, Win2, Wout2)
